# Initial kernel scaffold; baseline (speedup 1.0000x reference)
import functools

import jax
import jax.numpy as jnp
from jax import lax
from jax.experimental import pallas as pl
from jax.experimental.pallas import tpu as pltpu
from jax.experimental.pallas import tpu_sc as plsc

R0C = 5.0
NNODES = 10000
NEDGES = 160000
DA = 128

BATCH = 128
NBATCH = 80
EPAD = 16 * NBATCH * BATCH
ROWS_PER_TILE = NNODES // 16

BE = 1024


def _leaky(x):
    return jnp.where(x >= 0, x, 0.1 * x)


def _edge_body(rx, ry, rz, w0t, b0, w1t, b1, w2t, b2, w3t,
               p0, p1, p2, p3):
    x = rx[...]
    y = ry[...]
    z = rz[...]
    n2 = x * x + y * y + z * z
    r = jnp.sqrt(n2 + 1e-12)
    xr = r * (1.0 / R0C)
    centers = jnp.linspace(0.0, 1.0, 8, dtype=jnp.float32).reshape(1, 8)
    d = xr - centers
    enc = jnp.exp(-32.0 * d * d)
    h = jnp.dot(enc, w0t[...], preferred_element_type=jnp.float32) + b0[...]
    h = _leaky(jnp.dot(h, w1t[...], preferred_element_type=jnp.float32) + b1[...])
    h = _leaky(jnp.dot(h, w2t[...], preferred_element_type=jnp.float32) + b2[...])
    rad = jnp.dot(h, w3t[...], preferred_element_type=jnp.float32)
    eg = pl.program_id(0) * BE + lax.broadcasted_iota(jnp.int32, (BE, 1), 0)
    rad = jnp.where(eg < NEDGES, rad, 0.0)
    s = 7.0 / R0C
    inv = lax.rsqrt(1.0 + n2 * (s * s))
    sx = x * s * inv
    sy = y * s * inv
    sz = z * s * inv
    p0[...] = rad
    p1[...] = rad * sx
    p2[...] = rad * sy
    p3[...] = rad * sz


def _edge_mlp(rx, ry, rz, w0t, b0, w1t, b1, w2t, b2, w3t):
    grid = EPAD // BE
    col = pl.BlockSpec((BE, 1), lambda i: (i, 0))
    full = lambda a: pl.BlockSpec(a.shape, lambda i: (0,) * a.ndim)
    out = pl.BlockSpec((BE, DA), lambda i: (i, 0))
    return pl.pallas_call(
        _edge_body,
        grid=(grid,),
        in_specs=[col, col, col,
                  full(w0t), full(b0), full(w1t), full(b1),
                  full(w2t), full(b2), full(w3t)],
        out_specs=[out, out, out, out],
        out_shape=[jax.ShapeDtypeStruct((EPAD, DA), jnp.float32)] * 4,
    )(rx, ry, rz, w0t, b0, w1t, b1, w2t, b2, w3t)


def _sc_body(p0, p1, p2, p3, zeros_hbm, src3d,
             o0, o1, o2, o3, buf, idx, acc):
    c = lax.axis_index("c")
    s = lax.axis_index("s")
    pltpu.sync_copy(src3d.at[s], idx)
    nds = pl.ds(s * ROWS_PER_TILE, ROWS_PER_TILE)

    def do_chunk(p_hbm, out_hbm):
        pltpu.sync_copy(zeros_hbm.at[nds], acc.at[nds])
        plsc.subcore_barrier()

        def body(b, carry):
            base = (s * NBATCH + b) * BATCH
            pltpu.sync_copy(p_hbm.at[pl.ds(base, BATCH)], buf)
            pltpu.sync_copy(buf, acc.at[idx.at[b]], add=True)
            return carry

        lax.fori_loop(0, NBATCH, body, 0)
        plsc.subcore_barrier()
        pltpu.sync_copy(acc.at[nds], out_hbm.at[nds])
        plsc.subcore_barrier()

    @pl.when(c == 0)
    def _():
        do_chunk(p0, o0)
        do_chunk(p1, o1)

    @pl.when(c == 1)
    def _():
        do_chunk(p2, o2)
        do_chunk(p3, o3)


def _sc_scatter(p0, p1, p2, p3, zeros, src3d):
    mesh = plsc.VectorSubcoreMesh(core_axis_name="c", subcore_axis_name="s")
    fn = pl.kernel(
        _sc_body,
        out_type=[jax.ShapeDtypeStruct((NNODES, DA), jnp.float32)] * 4,
        mesh=mesh,
        scratch_types=[
            pltpu.VMEM((BATCH, DA), jnp.float32),
            pltpu.VMEM((NBATCH, BATCH), jnp.int32),
            pltpu.VMEM_SHARED((NNODES, DA), jnp.float32),
        ],
    )
    return fn(p0, p1, p2, p3, zeros, src3d)


def _readout_body(a1, a2, a3, wvt, y0, y1, y2):
    y0[...] = jnp.dot(a1[...], wvt[...], preferred_element_type=jnp.float32)
    y1[...] = jnp.dot(a2[...], wvt[...], preferred_element_type=jnp.float32)
    y2[...] = jnp.dot(a3[...], wvt[...], preferred_element_type=jnp.float32)


def _readout(a1, a2, a3, wvt):
    bn = 1000
    node = pl.BlockSpec((bn, DA), lambda i: (i, 0))
    wfull = pl.BlockSpec((DA, DA), lambda i: (0, 0))
    return pl.pallas_call(
        _readout_body,
        grid=(NNODES // bn,),
        in_specs=[node, node, node, wfull],
        out_specs=[node, node, node],
        out_shape=[jax.ShapeDtypeStruct((NNODES, DA), jnp.float32)] * 3,
    )(a1, a2, a3, wvt)


def kernel(graph, r_ij, W0, b0, W1, b1, W2, b2, W3, Wv):
    rpad = jnp.pad(r_ij, ((0, EPAD - NEDGES), (0, 0)))
    rx = rpad[:, 0:1]
    ry = rpad[:, 1:2]
    rz = rpad[:, 2:3]
    p0, p1, p2, p3 = _edge_mlp(
        rx, ry, rz,
        W0.T, b0.reshape(1, DA),
        W1.T, b1.reshape(1, DA),
        W2.T, b2.reshape(1, DA),
        W3.T)
    srcp = jnp.pad(graph[0], (0, EPAD - NEDGES)).reshape(16, NBATCH, BATCH)
    zeros = jnp.zeros((NNODES, DA), jnp.float32)
    a0, a1, a2, a3 = _sc_scatter(p0, p1, p2, p3, zeros, srcp)
    y0, y1, y2 = _readout(a1, a2, a3, Wv.T)
    return a0, jnp.stack([y0, y1, y2], axis=-1)


# speedup vs baseline: 20.7442x; 20.7442x over previous
import functools

import jax
import jax.numpy as jnp
from jax import lax
from jax.experimental import pallas as pl
from jax.experimental.pallas import tpu as pltpu
from jax.experimental.pallas import tpu_sc as plsc

R0C = 5.0
NNODES = 10000
NPAD = 10112
NEDGES = 160000
DA = 128

BATCH = 128
NBATCH = 80
EPAD = 16 * NBATCH * BATCH
ROWS_PER_TILE = NPAD // 16

BE = 1024


def _leaky(x):
    return jnp.where(x >= 0, x, 0.1 * x)


def _edge_body(rx, ry, rz, w0t, b0, w1t, b1, w2t, b2, w3t,
               p0, p1, p2, p3):
    x = rx[...]
    y = ry[...]
    z = rz[...]
    n2 = x * x + y * y + z * z
    r = jnp.sqrt(n2 + 1e-12)
    xr = r * (1.0 / R0C)
    centers = lax.broadcasted_iota(jnp.int32, (1, 8), 1).astype(jnp.float32) * (1.0 / 7.0)
    d = xr - centers
    enc = jnp.exp(-32.0 * d * d)
    h = jnp.dot(enc, w0t[...], preferred_element_type=jnp.float32) + b0[...]
    h = _leaky(jnp.dot(h, w1t[...], preferred_element_type=jnp.float32) + b1[...])
    h = _leaky(jnp.dot(h, w2t[...], preferred_element_type=jnp.float32) + b2[...])
    rad = jnp.dot(h, w3t[...], preferred_element_type=jnp.float32)
    eg = pl.program_id(0) * BE + lax.broadcasted_iota(jnp.int32, (BE, 1), 0)
    rad = jnp.where(eg < NEDGES, rad, 0.0)
    s = 7.0 / R0C
    inv = lax.rsqrt(1.0 + n2 * (s * s))
    sx = x * s * inv
    sy = y * s * inv
    sz = z * s * inv
    p0[...] = rad
    p1[...] = rad * sx
    p2[...] = rad * sy
    p3[...] = rad * sz


def _edge_mlp(rx, ry, rz, w0t, b0, w1t, b1, w2t, b2, w3t):
    grid = EPAD // BE
    col = pl.BlockSpec((BE, 1), lambda i: (i, 0))
    full = lambda a: pl.BlockSpec(a.shape, lambda i: (0,) * a.ndim)
    out = pl.BlockSpec((BE, DA), lambda i: (i, 0))
    return pl.pallas_call(
        _edge_body,
        grid=(grid,),
        in_specs=[col, col, col,
                  full(w0t), full(b0), full(w1t), full(b1),
                  full(w2t), full(b2), full(w3t)],
        out_specs=[out, out, out, out],
        out_shape=[jax.ShapeDtypeStruct((EPAD, DA), jnp.float32)] * 4,
    )(rx, ry, rz, w0t, b0, w1t, b1, w2t, b2, w3t)


def _sc_body(p0, p1, p2, p3, zeros_hbm, src3d,
             o0, o1, o2, o3, buf, idx, acc):
    c = lax.axis_index("c")
    s = lax.axis_index("s")
    pltpu.sync_copy(src3d.at[s], idx)
    nds = pl.ds(s * ROWS_PER_TILE, ROWS_PER_TILE)

    def do_chunk(p_hbm, out_hbm):
        pltpu.sync_copy(zeros_hbm.at[nds], acc.at[nds])
        plsc.subcore_barrier()

        def body(b, carry):
            base = (s * NBATCH + b) * BATCH
            pltpu.sync_copy(p_hbm.at[pl.ds(base, BATCH)], buf)
            pltpu.sync_copy(buf, acc.at[idx.at[b]], add=True)
            return carry

        lax.fori_loop(0, NBATCH, body, 0)
        plsc.subcore_barrier()
        pltpu.sync_copy(acc.at[nds], out_hbm.at[nds])
        plsc.subcore_barrier()

    @pl.when(c == 0)
    def _():
        do_chunk(p0, o0)
        do_chunk(p1, o1)

    @pl.when(c == 1)
    def _():
        do_chunk(p2, o2)
        do_chunk(p3, o3)


def _sc_scatter(p0, p1, p2, p3, zeros, src3d):
    mesh = plsc.VectorSubcoreMesh(core_axis_name="c", subcore_axis_name="s")
    fn = pl.kernel(
        _sc_body,
        out_type=[jax.ShapeDtypeStruct((NPAD, DA), jnp.float32)] * 4,
        mesh=mesh,
        scratch_types=[
            pltpu.VMEM((BATCH, DA), jnp.float32),
            pltpu.VMEM((NBATCH, BATCH), jnp.int32),
            pltpu.VMEM_SHARED((NPAD, DA), jnp.float32),
        ],
    )
    return fn(p0, p1, p2, p3, zeros, src3d)


def _readout_body(a1, a2, a3, wvt, y0, y1, y2):
    y0[...] = jnp.dot(a1[...], wvt[...], preferred_element_type=jnp.float32)
    y1[...] = jnp.dot(a2[...], wvt[...], preferred_element_type=jnp.float32)
    y2[...] = jnp.dot(a3[...], wvt[...], preferred_element_type=jnp.float32)


def _readout(a1, a2, a3, wvt):
    bn = 632
    node = pl.BlockSpec((bn, DA), lambda i: (i, 0))
    wfull = pl.BlockSpec((DA, DA), lambda i: (0, 0))
    return pl.pallas_call(
        _readout_body,
        grid=(NPAD // bn,),
        in_specs=[node, node, node, wfull],
        out_specs=[node, node, node],
        out_shape=[jax.ShapeDtypeStruct((NPAD, DA), jnp.float32)] * 3,
    )(a1, a2, a3, wvt)


def kernel(graph, r_ij, W0, b0, W1, b1, W2, b2, W3, Wv):
    rpad = jnp.pad(r_ij, ((0, EPAD - NEDGES), (0, 0)))
    rx = rpad[:, 0:1]
    ry = rpad[:, 1:2]
    rz = rpad[:, 2:3]
    p0, p1, p2, p3 = _edge_mlp(
        rx, ry, rz,
        W0.T, b0.reshape(1, DA),
        W1.T, b1.reshape(1, DA),
        W2.T, b2.reshape(1, DA),
        W3.T)
    srcp = jnp.pad(graph[0], (0, EPAD - NEDGES)).reshape(16, NBATCH, BATCH)
    zeros = jnp.zeros((NPAD, DA), jnp.float32)
    a0, a1, a2, a3 = _sc_scatter(p0, p1, p2, p3, zeros, srcp)
    y0, y1, y2 = _readout(a1, a2, a3, Wv.T)
    out_v = jnp.stack([y0, y1, y2], axis=-1)[:NNODES]
    return a0[:NNODES], out_v


# speedup vs baseline: 23.4185x; 1.1289x over previous
import functools

import jax
import jax.numpy as jnp
from jax import lax
from jax.experimental import pallas as pl
from jax.experimental.pallas import tpu as pltpu
from jax.experimental.pallas import tpu_sc as plsc

R0C = 5.0
NNODES = 10000
NPAD = 10112
NEDGES = 160000
DA = 128

BATCH = 128
NBATCH = 80
EPAD = 16 * NBATCH * BATCH
ROWS_PER_TILE = NPAD // 16

BE = 1024


def _leaky(x):
    return jnp.maximum(x, 0.1 * x)


def _edge_body(rx, ry, rz, w0t, b0, w1t, b1, w2t, b2, w3t,
               p0, p1, p2, p3):
    x = rx[...]
    y = ry[...]
    z = rz[...]
    n2 = x * x + y * y + z * z
    r = jnp.sqrt(n2 + 1e-12)
    xr = r * (1.0 / R0C)
    centers = lax.broadcasted_iota(jnp.int32, (1, 8), 1).astype(jnp.float32) * (1.0 / 7.0)
    d = xr - centers
    enc = jnp.exp(-32.0 * d * d)
    h = jnp.dot(enc, w0t[...], preferred_element_type=jnp.float32) + b0[...]
    h = _leaky(jnp.dot(h, w1t[...], preferred_element_type=jnp.float32) + b1[...])
    h = _leaky(jnp.dot(h, w2t[...], preferred_element_type=jnp.float32) + b2[...])
    rad = jnp.dot(h, w3t[...], preferred_element_type=jnp.float32)
    s = 7.0 / R0C
    inv = lax.rsqrt(1.0 + n2 * (s * s))
    sx = x * s * inv
    sy = y * s * inv
    sz = z * s * inv
    p0[...] = rad
    p1[...] = rad * sx
    p2[...] = rad * sy
    p3[...] = rad * sz


def _edge_mlp(rx, ry, rz, w0t, b0, w1t, b1, w2t, b2, w3t):
    grid = EPAD // BE
    col = pl.BlockSpec((BE, 1), lambda i: (i, 0))
    full = lambda a: pl.BlockSpec(a.shape, lambda i: (0,) * a.ndim)
    out = pl.BlockSpec((BE, DA), lambda i: (i, 0))
    return pl.pallas_call(
        _edge_body,
        grid=(grid,),
        in_specs=[col, col, col,
                  full(w0t), full(b0), full(w1t), full(b1),
                  full(w2t), full(b2), full(w3t)],
        out_specs=[out, out, out, out],
        out_shape=[jax.ShapeDtypeStruct((EPAD, DA), jnp.float32)] * 4,
    )(rx, ry, rz, w0t, b0, w1t, b1, w2t, b2, w3t)


def _sc_body(p0, p1, p2, p3, zeros_hbm, src3d,
             o0, o1, o2, o3, buf_a, buf_b, idx, acc,
             gs_a, gs_b, ss_a, ss_b):
    c = lax.axis_index("c")
    s = lax.axis_index("s")
    pltpu.sync_copy(src3d.at[s], idx)
    nds = pl.ds(s * ROWS_PER_TILE, ROWS_PER_TILE)

    def do_chunk(p_hbm, out_hbm):
        def batch_ds(b):
            return pl.ds((s * NBATCH + b) * BATCH, BATCH)

        def g_start(b, buf, sem):
            pltpu.async_copy(p_hbm.at[batch_ds(b)], buf, sem)

        def g_wait(buf, sem):
            pltpu.make_async_copy(p_hbm.at[batch_ds(0)], buf, sem).wait()

        def s_start(b, buf, sem):
            pltpu.async_copy(buf, acc.at[idx.at[b]], sem, add=True)

        def s_wait(buf, sem):
            pltpu.make_async_copy(buf, acc.at[idx.at[0]], sem).wait()

        g_start(0, buf_a, gs_a)
        pltpu.sync_copy(zeros_hbm.at[nds], acc.at[nds])
        plsc.subcore_barrier()

        def body(g, carry):
            b0 = 2 * g
            b1 = b0 + 1
            g_wait(buf_a, gs_a)

            @pl.when(g > 0)
            def _():
                s_wait(buf_b, ss_b)

            g_start(b1, buf_b, gs_b)
            s_start(b0, buf_a, ss_a)
            g_wait(buf_b, gs_b)
            s_wait(buf_a, ss_a)

            @pl.when(g < NBATCH // 2 - 1)
            def _():
                g_start(b0 + 2, buf_a, gs_a)

            s_start(b1, buf_b, ss_b)
            return carry

        lax.fori_loop(0, NBATCH // 2, body, 0)
        s_wait(buf_b, ss_b)
        plsc.subcore_barrier()
        pltpu.sync_copy(acc.at[nds], out_hbm.at[nds])
        plsc.subcore_barrier()

    @pl.when(c == 0)
    def _():
        do_chunk(p0, o0)
        do_chunk(p1, o1)

    @pl.when(c == 1)
    def _():
        do_chunk(p2, o2)
        do_chunk(p3, o3)


def _sc_scatter(p0, p1, p2, p3, zeros, src3d):
    mesh = plsc.VectorSubcoreMesh(core_axis_name="c", subcore_axis_name="s")
    fn = pl.kernel(
        _sc_body,
        out_type=[jax.ShapeDtypeStruct((NPAD, DA), jnp.float32)] * 4,
        mesh=mesh,
        scratch_types=[
            pltpu.VMEM((BATCH, DA), jnp.float32),
            pltpu.VMEM((BATCH, DA), jnp.float32),
            pltpu.VMEM((NBATCH, BATCH), jnp.int32),
            pltpu.VMEM_SHARED((NPAD, DA), jnp.float32),
            pltpu.SemaphoreType.DMA,
            pltpu.SemaphoreType.DMA,
            pltpu.SemaphoreType.DMA,
            pltpu.SemaphoreType.DMA,
        ],
    )
    return fn(p0, p1, p2, p3, zeros, src3d)


def _readout_body(a1, a2, a3, wvt, y0, y1, y2):
    y0[...] = jnp.dot(a1[...], wvt[...], preferred_element_type=jnp.float32)
    y1[...] = jnp.dot(a2[...], wvt[...], preferred_element_type=jnp.float32)
    y2[...] = jnp.dot(a3[...], wvt[...], preferred_element_type=jnp.float32)


def _readout(a1, a2, a3, wvt):
    bn = 632
    node = pl.BlockSpec((bn, DA), lambda i: (i, 0))
    wfull = pl.BlockSpec((DA, DA), lambda i: (0, 0))
    return pl.pallas_call(
        _readout_body,
        grid=(NPAD // bn,),
        in_specs=[node, node, node, wfull],
        out_specs=[node, node, node],
        out_shape=[jax.ShapeDtypeStruct((NPAD, DA), jnp.float32)] * 3,
    )(a1, a2, a3, wvt)


def kernel(graph, r_ij, W0, b0, W1, b1, W2, b2, W3, Wv):
    rpad = jnp.pad(r_ij, ((0, EPAD - NEDGES), (0, 0)))
    rx = rpad[:, 0:1]
    ry = rpad[:, 1:2]
    rz = rpad[:, 2:3]
    p0, p1, p2, p3 = _edge_mlp(
        rx, ry, rz,
        W0.T, b0.reshape(1, DA),
        W1.T, b1.reshape(1, DA),
        W2.T, b2.reshape(1, DA),
        W3.T)
    srcp = jnp.pad(graph[0], (0, EPAD - NEDGES),
                   constant_values=NNODES).reshape(16, NBATCH, BATCH)
    zeros = jnp.zeros((NPAD, DA), jnp.float32)
    a0, a1, a2, a3 = _sc_scatter(p0, p1, p2, p3, zeros, srcp)
    y0, y1, y2 = _readout(a1, a2, a3, Wv.T)
    out_v = jnp.stack([y0, y1, y2], axis=-1)[:NNODES]
    return a0[:NNODES], out_v


# speedup vs baseline: 36.3219x; 1.5510x over previous
import functools

import jax
import jax.numpy as jnp
from jax import lax
from jax.experimental import pallas as pl
from jax.experimental.pallas import tpu as pltpu
from jax.experimental.pallas import tpu_sc as plsc

R0C = 5.0
NNODES = 10000
NPAD = 10112
NEDGES = 160000
DA = 128

BATCH = 128
NBATCH = 80
EPAD = 16 * NBATCH * BATCH
ROWS_PER_TILE = NPAD // 16

BE = 1024


def _leaky(x):
    return jnp.maximum(x, 0.1 * x)


def _edge_body(rt, w0t, b0, w1t, b1, w2t, b2, w3t,
               p0, p1, p2, p3):
    x = rt[0:1, :]
    y = rt[1:2, :]
    z = rt[2:3, :]
    n2 = x * x + y * y + z * z
    xr = jnp.sqrt(n2 + 1e-12) * (1.0 / R0C)
    centers = lax.broadcasted_iota(jnp.int32, (8, 1), 0).astype(jnp.float32) * (1.0 / 7.0)
    d = xr - centers
    enc = jnp.transpose(jnp.exp(-32.0 * d * d), (1, 0))
    h = jnp.dot(enc, w0t[...], preferred_element_type=jnp.float32) + b0[...]
    h = _leaky(jnp.dot(h, w1t[...], preferred_element_type=jnp.float32) + b1[...])
    h = _leaky(jnp.dot(h, w2t[...], preferred_element_type=jnp.float32) + b2[...])
    rad = jnp.dot(h, w3t[...], preferred_element_type=jnp.float32)
    s = 7.0 / R0C
    inv = lax.rsqrt(1.0 + n2 * (s * s))
    srow = jnp.concatenate(
        [x * (s * inv), y * (s * inv), z * (s * inv),
         jnp.zeros((5, x.shape[1]), jnp.float32)], axis=0)
    scol = jnp.transpose(srow, (1, 0))
    p0[...] = rad
    p1[...] = rad * scol[:, 0:1]
    p2[...] = rad * scol[:, 1:2]
    p3[...] = rad * scol[:, 2:3]


def _edge_mlp(rt, w0t, b0, w1t, b1, w2t, b2, w3t):
    grid = EPAD // BE
    rspec = pl.BlockSpec((3, BE), lambda i: (0, i))
    full = lambda a: pl.BlockSpec(a.shape, lambda i: (0,) * a.ndim)
    out = pl.BlockSpec((BE, DA), lambda i: (i, 0))
    return pl.pallas_call(
        _edge_body,
        grid=(grid,),
        in_specs=[rspec,
                  full(w0t), full(b0), full(w1t), full(b1),
                  full(w2t), full(b2), full(w3t)],
        out_specs=[out, out, out, out],
        out_shape=[jax.ShapeDtypeStruct((EPAD, DA), jnp.float32)] * 4,
    )(rt, w0t, b0, w1t, b1, w2t, b2, w3t)


def _sc_body(p0, p1, p2, p3, zeros_hbm, src3d,
             o0, o1, o2, o3, buf_a, buf_b, idx, acc,
             gs_a, gs_b, ss_a, ss_b):
    c = lax.axis_index("c")
    s = lax.axis_index("s")
    pltpu.sync_copy(src3d.at[s], idx)
    nds = pl.ds(s * ROWS_PER_TILE, ROWS_PER_TILE)

    def do_chunk(p_hbm, out_hbm):
        def batch_ds(b):
            return pl.ds((s * NBATCH + b) * BATCH, BATCH)

        def g_start(b, buf, sem):
            pltpu.async_copy(p_hbm.at[batch_ds(b)], buf, sem)

        def g_wait(buf, sem):
            pltpu.make_async_copy(p_hbm.at[batch_ds(0)], buf, sem).wait()

        def s_start(b, buf, sem):
            pltpu.async_copy(buf, acc.at[idx.at[b]], sem, add=True)

        def s_wait(buf, sem):
            pltpu.make_async_copy(buf, acc.at[idx.at[0]], sem).wait()

        g_start(0, buf_a, gs_a)
        pltpu.sync_copy(zeros_hbm.at[nds], acc.at[nds])
        plsc.subcore_barrier()

        def body(g, carry):
            b0 = 2 * g
            b1 = b0 + 1
            g_wait(buf_a, gs_a)

            @pl.when(g > 0)
            def _():
                s_wait(buf_b, ss_b)

            g_start(b1, buf_b, gs_b)
            s_start(b0, buf_a, ss_a)
            g_wait(buf_b, gs_b)
            s_wait(buf_a, ss_a)

            @pl.when(g < NBATCH // 2 - 1)
            def _():
                g_start(b0 + 2, buf_a, gs_a)

            s_start(b1, buf_b, ss_b)
            return carry

        lax.fori_loop(0, NBATCH // 2, body, 0)
        s_wait(buf_b, ss_b)
        plsc.subcore_barrier()
        pltpu.sync_copy(acc.at[nds], out_hbm.at[nds])
        plsc.subcore_barrier()

    @pl.when(c == 0)
    def _():
        do_chunk(p0, o0)
        do_chunk(p1, o1)

    @pl.when(c == 1)
    def _():
        do_chunk(p2, o2)
        do_chunk(p3, o3)


def _sc_scatter(p0, p1, p2, p3, zeros, src3d):
    mesh = plsc.VectorSubcoreMesh(core_axis_name="c", subcore_axis_name="s")
    fn = pl.kernel(
        _sc_body,
        out_type=[jax.ShapeDtypeStruct((NPAD, DA), jnp.float32)] * 4,
        mesh=mesh,
        scratch_types=[
            pltpu.VMEM((BATCH, DA), jnp.float32),
            pltpu.VMEM((BATCH, DA), jnp.float32),
            pltpu.VMEM((NBATCH, BATCH), jnp.int32),
            pltpu.VMEM_SHARED((NPAD, DA), jnp.float32),
            pltpu.SemaphoreType.DMA,
            pltpu.SemaphoreType.DMA,
            pltpu.SemaphoreType.DMA,
            pltpu.SemaphoreType.DMA,
        ],
    )
    return fn(p0, p1, p2, p3, zeros, src3d)


def _readout_body(a1, a2, a3, wvt, y0, y1, y2):
    y0[...] = jnp.dot(a1[...], wvt[...], preferred_element_type=jnp.float32)
    y1[...] = jnp.dot(a2[...], wvt[...], preferred_element_type=jnp.float32)
    y2[...] = jnp.dot(a3[...], wvt[...], preferred_element_type=jnp.float32)


def _readout(a1, a2, a3, wvt):
    bn = 632
    node = pl.BlockSpec((bn, DA), lambda i: (i, 0))
    wfull = pl.BlockSpec((DA, DA), lambda i: (0, 0))
    return pl.pallas_call(
        _readout_body,
        grid=(NPAD // bn,),
        in_specs=[node, node, node, wfull],
        out_specs=[node, node, node],
        out_shape=[jax.ShapeDtypeStruct((NPAD, DA), jnp.float32)] * 3,
    )(a1, a2, a3, wvt)


def kernel(graph, r_ij, W0, b0, W1, b1, W2, b2, W3, Wv):
    rt = jnp.pad(r_ij.T, ((0, 0), (0, EPAD - NEDGES)))
    p0, p1, p2, p3 = _edge_mlp(
        rt,
        W0.T, b0.reshape(1, DA),
        W1.T, b1.reshape(1, DA),
        W2.T, b2.reshape(1, DA),
        W3.T)
    srcp = jnp.pad(graph[0], (0, EPAD - NEDGES),
                   constant_values=NNODES).reshape(16, NBATCH, BATCH)
    zeros = jnp.zeros((NPAD, DA), jnp.float32)
    a0, a1, a2, a3 = _sc_scatter(p0, p1, p2, p3, zeros, srcp)
    y0, y1, y2 = _readout(a1, a2, a3, Wv.T)
    out_v = jnp.stack([y0, y1, y2], axis=-1)[:NNODES]
    return a0[:NNODES], out_v


# speedup vs baseline: 39.9376x; 1.0995x over previous
import functools

import jax
import jax.numpy as jnp
from jax import lax
from jax.experimental import pallas as pl
from jax.experimental.pallas import tpu as pltpu
from jax.experimental.pallas import tpu_sc as plsc

R0C = 5.0
NNODES = 10000
NPAD = 10112
NEDGES = 160000
DA = 128

BATCH = 128
NBATCH = 80
EPAD = 16 * NBATCH * BATCH
ROWS_PER_TILE = NPAD // 16

BE = 2048


def _leaky(x):
    return jnp.maximum(x, 0.1 * x)


def _edge_body(rt, w0t, b0, w1t, b1, w2t, b2, w3t,
               p0, p1, p2, p3):
    x = rt[0:1, :]
    y = rt[1:2, :]
    z = rt[2:3, :]
    n2 = x * x + y * y + z * z
    xr = jnp.sqrt(n2 + 1e-12) * (1.0 / R0C)
    centers = lax.broadcasted_iota(jnp.int32, (8, 1), 0).astype(jnp.float32) * (1.0 / 7.0)
    d = xr - centers
    enc = jnp.transpose(jnp.exp(-32.0 * d * d), (1, 0))
    h = jnp.dot(enc, w0t[...], preferred_element_type=jnp.float32) + b0[...]
    h = _leaky(jnp.dot(h, w1t[...], preferred_element_type=jnp.float32) + b1[...])
    h = _leaky(jnp.dot(h, w2t[...], preferred_element_type=jnp.float32) + b2[...])
    rad = jnp.dot(h, w3t[...], preferred_element_type=jnp.float32)
    s = 7.0 / R0C
    inv = lax.rsqrt(1.0 + n2 * (s * s))
    srow = jnp.concatenate(
        [x * (s * inv), y * (s * inv), z * (s * inv),
         jnp.zeros((5, x.shape[1]), jnp.float32)], axis=0)
    scol = jnp.transpose(srow, (1, 0))
    p0[...] = rad
    p1[...] = rad * scol[:, 0:1]
    p2[...] = rad * scol[:, 1:2]
    p3[...] = rad * scol[:, 2:3]


def _edge_mlp(rt, w0t, b0, w1t, b1, w2t, b2, w3t):
    grid = EPAD // BE
    rspec = pl.BlockSpec((3, BE), lambda i: (0, i))
    full = lambda a: pl.BlockSpec(a.shape, lambda i: (0,) * a.ndim)
    out = pl.BlockSpec((BE, DA), lambda i: (i, 0))
    return pl.pallas_call(
        _edge_body,
        grid=(grid,),
        in_specs=[rspec,
                  full(w0t), full(b0), full(w1t), full(b1),
                  full(w2t), full(b2), full(w3t)],
        out_specs=[out, out, out, out],
        out_shape=[jax.ShapeDtypeStruct((EPAD, DA), jnp.float32)] * 4,
    )(rt, w0t, b0, w1t, b1, w2t, b2, w3t)


def _sc_body(p0, p1, p2, p3, zeros_hbm, src3d,
             o0, o1, o2, o3, buf_a, buf_b, idx, acc,
             gs_a, gs_b, ss_a, ss_b):
    c = lax.axis_index("c")
    s = lax.axis_index("s")
    pltpu.sync_copy(src3d.at[s], idx)
    nds = pl.ds(s * ROWS_PER_TILE, ROWS_PER_TILE)

    def do_chunk(p_hbm, out_hbm):
        def batch_ds(b):
            return pl.ds((s * NBATCH + b) * BATCH, BATCH)

        def g_start(b, buf, sem):
            pltpu.async_copy(p_hbm.at[batch_ds(b)], buf, sem)

        def g_wait(buf, sem):
            pltpu.make_async_copy(p_hbm.at[batch_ds(0)], buf, sem).wait()

        def s_start(b, buf, sem):
            pltpu.async_copy(buf, acc.at[idx.at[b]], sem, add=True)

        def s_wait(buf, sem):
            pltpu.make_async_copy(buf, acc.at[idx.at[0]], sem).wait()

        g_start(0, buf_a, gs_a)
        pltpu.sync_copy(zeros_hbm.at[nds], acc.at[nds])
        plsc.subcore_barrier()

        def body(g, carry):
            b0 = 2 * g
            b1 = b0 + 1
            g_wait(buf_a, gs_a)

            @pl.when(g > 0)
            def _():
                s_wait(buf_b, ss_b)

            g_start(b1, buf_b, gs_b)
            s_start(b0, buf_a, ss_a)
            g_wait(buf_b, gs_b)
            s_wait(buf_a, ss_a)

            @pl.when(g < NBATCH // 2 - 1)
            def _():
                g_start(b0 + 2, buf_a, gs_a)

            s_start(b1, buf_b, ss_b)
            return carry

        lax.fori_loop(0, NBATCH // 2, body, 0)
        s_wait(buf_b, ss_b)
        plsc.subcore_barrier()
        pltpu.sync_copy(acc.at[nds], out_hbm.at[nds])
        plsc.subcore_barrier()

    @pl.when(c == 0)
    def _():
        do_chunk(p0, o0)
        do_chunk(p1, o1)

    @pl.when(c == 1)
    def _():
        do_chunk(p2, o2)
        do_chunk(p3, o3)


def _sc_scatter(p0, p1, p2, p3, zeros, src3d):
    mesh = plsc.VectorSubcoreMesh(core_axis_name="c", subcore_axis_name="s")
    fn = pl.kernel(
        _sc_body,
        out_type=[jax.ShapeDtypeStruct((NPAD, DA), jnp.float32)] * 4,
        mesh=mesh,
        scratch_types=[
            pltpu.VMEM((BATCH, DA), jnp.float32),
            pltpu.VMEM((BATCH, DA), jnp.float32),
            pltpu.VMEM((NBATCH, BATCH), jnp.int32),
            pltpu.VMEM_SHARED((NPAD, DA), jnp.float32),
            pltpu.SemaphoreType.DMA,
            pltpu.SemaphoreType.DMA,
            pltpu.SemaphoreType.DMA,
            pltpu.SemaphoreType.DMA,
        ],
    )
    return fn(p0, p1, p2, p3, zeros, src3d)


def _readout_body(a1, a2, a3, wvt, y0, y1, y2):
    y0[...] = jnp.dot(a1[...], wvt[...], preferred_element_type=jnp.float32)
    y1[...] = jnp.dot(a2[...], wvt[...], preferred_element_type=jnp.float32)
    y2[...] = jnp.dot(a3[...], wvt[...], preferred_element_type=jnp.float32)


def _readout(a1, a2, a3, wvt):
    bn = 632
    node = pl.BlockSpec((bn, DA), lambda i: (i, 0))
    wfull = pl.BlockSpec((DA, DA), lambda i: (0, 0))
    return pl.pallas_call(
        _readout_body,
        grid=(NPAD // bn,),
        in_specs=[node, node, node, wfull],
        out_specs=[node, node, node],
        out_shape=[jax.ShapeDtypeStruct((NPAD, DA), jnp.float32)] * 3,
    )(a1, a2, a3, wvt)


def kernel(graph, r_ij, W0, b0, W1, b1, W2, b2, W3, Wv):
    rt = jnp.pad(r_ij.T, ((0, 0), (0, EPAD - NEDGES)))
    p0, p1, p2, p3 = _edge_mlp(
        rt,
        W0.T, b0.reshape(1, DA),
        W1.T, b1.reshape(1, DA),
        W2.T, b2.reshape(1, DA),
        W3.T)
    srcp = jnp.pad(graph[0], (0, EPAD - NEDGES),
                   constant_values=NNODES).reshape(16, NBATCH, BATCH)
    zeros = jnp.zeros((NPAD, DA), jnp.float32)
    a0, a1, a2, a3 = _sc_scatter(p0, p1, p2, p3, zeros, srcp)
    y0, y1, y2 = _readout(a1, a2, a3, Wv.T)
    out_v = jnp.stack([y0, y1, y2], axis=-1)[:NNODES]
    return a0[:NNODES], out_v


# speedup vs baseline: 40.1296x; 1.0048x over previous
import jax
import jax.numpy as jnp
from jax import lax
from jax.experimental import pallas as pl
from jax.experimental.pallas import tpu as pltpu
from jax.experimental.pallas import tpu_sc as plsc

R0C = 5.0
NNODES = 10000
NPAD = 10112
NEDGES = 160000
DA = 128

BATCH = 128
NBATCH = 40
EPAD_H = 16 * NBATCH * BATCH
ROWS_PER_TILE = NPAD // 16

BE = 2048


def _leaky(x):
    return jnp.maximum(x, 0.1 * x)


def _edge_body(rt, w0t, b0, w1t, b1, w2t, b2, w3t,
               p0, p1, p2, p3):
    x = rt[0:1, :]
    y = rt[1:2, :]
    z = rt[2:3, :]
    n2 = x * x + y * y + z * z
    xr = jnp.sqrt(n2 + 1e-12) * (1.0 / R0C)
    centers = lax.broadcasted_iota(jnp.int32, (8, 1), 0).astype(jnp.float32) * (1.0 / 7.0)
    d = xr - centers
    enc = jnp.transpose(jnp.exp(-32.0 * d * d), (1, 0))
    h = jnp.dot(enc, w0t[...], preferred_element_type=jnp.float32) + b0[...]
    h = _leaky(jnp.dot(h, w1t[...], preferred_element_type=jnp.float32) + b1[...])
    h = _leaky(jnp.dot(h, w2t[...], preferred_element_type=jnp.float32) + b2[...])
    rad = jnp.dot(h, w3t[...], preferred_element_type=jnp.float32)
    s = 7.0 / R0C
    inv = lax.rsqrt(1.0 + n2 * (s * s))
    srow = jnp.concatenate(
        [x * (s * inv), y * (s * inv), z * (s * inv),
         jnp.zeros((5, x.shape[1]), jnp.float32)], axis=0)
    scol = jnp.transpose(srow, (1, 0))
    p0[...] = rad
    p1[...] = rad * scol[:, 0:1]
    p2[...] = rad * scol[:, 1:2]
    p3[...] = rad * scol[:, 2:3]


def _edge_mlp(rt, w0t, b0, w1t, b1, w2t, b2, w3t):
    grid = EPAD_H // BE
    rspec = pl.BlockSpec((3, BE), lambda i: (0, i))
    full = lambda a: pl.BlockSpec(a.shape, lambda i: (0,) * a.ndim)
    out = pl.BlockSpec((BE, DA), lambda i: (i, 0))
    return pl.pallas_call(
        _edge_body,
        grid=(grid,),
        in_specs=[rspec,
                  full(w0t), full(b0), full(w1t), full(b1),
                  full(w2t), full(b2), full(w3t)],
        out_specs=[out, out, out, out],
        out_shape=[jax.ShapeDtypeStruct((EPAD_H, DA), jnp.float32)] * 4,
    )(rt, w0t, b0, w1t, b1, w2t, b2, w3t)


def _sc_body(p0, p1, p2, p3, zeros_hbm, src3d,
             o0, o1, o2, o3, buf_a, buf_b, idx, acc,
             gs_a, gs_b, ss_a, ss_b):
    c = lax.axis_index("c")
    s = lax.axis_index("s")
    pltpu.sync_copy(src3d.at[s], idx)
    nds = pl.ds(s * ROWS_PER_TILE, ROWS_PER_TILE)

    def do_chunk(p_hbm, out_hbm):
        def batch_ds(b):
            return pl.ds((s * NBATCH + b) * BATCH, BATCH)

        def g_start(b, buf, sem):
            pltpu.async_copy(p_hbm.at[batch_ds(b)], buf, sem)

        def g_wait(buf, sem):
            pltpu.make_async_copy(p_hbm.at[batch_ds(0)], buf, sem).wait()

        def s_start(b, buf, sem):
            pltpu.async_copy(buf, acc.at[idx.at[b]], sem, add=True)

        def s_wait(buf, sem):
            pltpu.make_async_copy(buf, acc.at[idx.at[0]], sem).wait()

        g_start(0, buf_a, gs_a)
        pltpu.sync_copy(zeros_hbm.at[nds], acc.at[nds])
        plsc.subcore_barrier()

        def body(g, carry):
            b0 = 2 * g
            b1 = b0 + 1
            g_wait(buf_a, gs_a)

            @pl.when(g > 0)
            def _():
                s_wait(buf_b, ss_b)

            g_start(b1, buf_b, gs_b)
            s_start(b0, buf_a, ss_a)
            g_wait(buf_b, gs_b)
            s_wait(buf_a, ss_a)

            @pl.when(g < NBATCH // 2 - 1)
            def _():
                g_start(b0 + 2, buf_a, gs_a)

            s_start(b1, buf_b, ss_b)
            return carry

        lax.fori_loop(0, NBATCH // 2, body, 0)
        s_wait(buf_b, ss_b)
        plsc.subcore_barrier()
        pltpu.sync_copy(acc.at[nds], out_hbm.at[nds])
        plsc.subcore_barrier()

    @pl.when(c == 0)
    def _():
        do_chunk(p0, o0)
        do_chunk(p1, o1)

    @pl.when(c == 1)
    def _():
        do_chunk(p2, o2)
        do_chunk(p3, o3)


def _sc_scatter(p0, p1, p2, p3, zeros, src3d):
    mesh = plsc.VectorSubcoreMesh(core_axis_name="c", subcore_axis_name="s")
    fn = pl.kernel(
        _sc_body,
        out_type=[jax.ShapeDtypeStruct((NPAD, DA), jnp.float32)] * 4,
        mesh=mesh,
        scratch_types=[
            pltpu.VMEM((BATCH, DA), jnp.float32),
            pltpu.VMEM((BATCH, DA), jnp.float32),
            pltpu.VMEM((NBATCH, BATCH), jnp.int32),
            pltpu.VMEM_SHARED((NPAD, DA), jnp.float32),
            pltpu.SemaphoreType.DMA,
            pltpu.SemaphoreType.DMA,
            pltpu.SemaphoreType.DMA,
            pltpu.SemaphoreType.DMA,
        ],
    )
    return fn(p0, p1, p2, p3, zeros, src3d)


def _readout_body(a0u, a0v, a1u, a1v, a2u, a2v, a3u, a3v, wvt,
                  aa, y0, y1, y2):
    aa[...] = a0u[...] + a0v[...]
    y0[...] = jnp.dot(a1u[...] + a1v[...], wvt[...],
                      preferred_element_type=jnp.float32)
    y1[...] = jnp.dot(a2u[...] + a2v[...], wvt[...],
                      preferred_element_type=jnp.float32)
    y2[...] = jnp.dot(a3u[...] + a3v[...], wvt[...],
                      preferred_element_type=jnp.float32)


def _readout(accs_u, accs_v, wvt):
    bn = 632
    node = pl.BlockSpec((bn, DA), lambda i: (i, 0))
    wfull = pl.BlockSpec((DA, DA), lambda i: (0, 0))
    args = []
    for u, v in zip(accs_u, accs_v):
        args += [u, v]
    return pl.pallas_call(
        _readout_body,
        grid=(NPAD // bn,),
        in_specs=[node] * 8 + [wfull],
        out_specs=[node, node, node, node],
        out_shape=[jax.ShapeDtypeStruct((NPAD, DA), jnp.float32)] * 4,
    )(*args, wvt)


def kernel(graph, r_ij, W0, b0, W1, b1, W2, b2, W3, Wv):
    rt = r_ij.T
    rt1 = rt[:, :EPAD_H]
    rt2 = jnp.pad(rt[:, EPAD_H:], ((0, 0), (0, 2 * EPAD_H - NEDGES)))
    src = graph[0]
    src1 = src[:EPAD_H].reshape(16, NBATCH, BATCH)
    src2 = jnp.pad(src[EPAD_H:], (0, 2 * EPAD_H - NEDGES),
                   constant_values=NNODES).reshape(16, NBATCH, BATCH)
    wargs = (W0.T, b0.reshape(1, DA), W1.T, b1.reshape(1, DA),
             W2.T, b2.reshape(1, DA), W3.T)
    zeros = jnp.zeros((NPAD, DA), jnp.float32)
    pu = _edge_mlp(rt1, *wargs)
    au = _sc_scatter(*pu, zeros, src1)
    pv = _edge_mlp(rt2, *wargs)
    av = _sc_scatter(*pv, zeros, src2)
    aa, y0, y1, y2 = _readout(au, av, Wv.T)
    out_v = jnp.stack([y0, y1, y2], axis=-1)[:NNODES]
    return aa[:NNODES], out_v


# speedup vs baseline: 43.6301x; 1.0872x over previous
import jax
import jax.numpy as jnp
from jax import lax
from jax.experimental import pallas as pl
from jax.experimental.pallas import tpu as pltpu
from jax.experimental.pallas import tpu_sc as plsc

R0C = 5.0
NNODES = 10000
NPAD = 10112
NEDGES = 160000
DA = 128

BATCH = 80
NBATCH = 64
EPAD_H = 16 * NBATCH * BATCH
ROWS_PER_TILE = NPAD // 16

BE = 2048


def _leaky(x):
    return jnp.maximum(x, 0.1 * x)


def _edge_body(rt, w0t, b0, w1t, b1, w2t, b2, w3t,
               p0, p1, p2, p3):
    x = rt[0:1, :]
    y = rt[1:2, :]
    z = rt[2:3, :]
    n2 = x * x + y * y + z * z
    xr = jnp.sqrt(n2 + 1e-12) * (1.0 / R0C)
    centers = lax.broadcasted_iota(jnp.int32, (8, 1), 0).astype(jnp.float32) * (1.0 / 7.0)
    d = xr - centers
    enc = jnp.transpose(jnp.exp(-32.0 * d * d), (1, 0))
    h = jnp.dot(enc, w0t[...], preferred_element_type=jnp.float32) + b0[...]
    h = _leaky(jnp.dot(h, w1t[...], preferred_element_type=jnp.float32) + b1[...])
    h = _leaky(jnp.dot(h, w2t[...], preferred_element_type=jnp.float32) + b2[...])
    rad = jnp.dot(h, w3t[...], preferred_element_type=jnp.float32)
    s = 7.0 / R0C
    inv = lax.rsqrt(1.0 + n2 * (s * s))
    srow = jnp.concatenate(
        [x * (s * inv), y * (s * inv), z * (s * inv),
         jnp.zeros((5, x.shape[1]), jnp.float32)], axis=0)
    scol = jnp.transpose(srow, (1, 0))
    p0[...] = rad
    p1[...] = rad * scol[:, 0:1]
    p2[...] = rad * scol[:, 1:2]
    p3[...] = rad * scol[:, 2:3]


def _edge_mlp(rt, w0t, b0, w1t, b1, w2t, b2, w3t):
    grid = EPAD_H // BE
    rspec = pl.BlockSpec((3, BE), lambda i: (0, i))
    full = lambda a: pl.BlockSpec(a.shape, lambda i: (0,) * a.ndim)
    out = pl.BlockSpec((BE, DA), lambda i: (i, 0))
    return pl.pallas_call(
        _edge_body,
        grid=(grid,),
        in_specs=[rspec,
                  full(w0t), full(b0), full(w1t), full(b1),
                  full(w2t), full(b2), full(w3t)],
        out_specs=[out, out, out, out],
        out_shape=[jax.ShapeDtypeStruct((EPAD_H, DA), jnp.float32)] * 4,
    )(rt, w0t, b0, w1t, b1, w2t, b2, w3t)


NRING = 4


def _sc_body(p0, p1, p2, p3, zeros_hbm, src3d,
             o0, o1, o2, o3, buf0, buf1, buf2, buf3, idx, acc,
             gs0, gs1, gs2, gs3, ss0, ss1, ss2, ss3):
    bufs = (buf0, buf1, buf2, buf3)
    gsems = (gs0, gs1, gs2, gs3)
    ssems = (ss0, ss1, ss2, ss3)
    c = lax.axis_index("c")
    s = lax.axis_index("s")
    pltpu.sync_copy(src3d.at[s], idx)
    nds = pl.ds(s * ROWS_PER_TILE, ROWS_PER_TILE)
    ngrp = NBATCH // NRING

    def do_chunk(p_hbm, out_hbm):
        def batch_ds(b):
            return pl.ds((s * NBATCH + b) * BATCH, BATCH)

        def g_start(b, j):
            pltpu.async_copy(p_hbm.at[batch_ds(b)], bufs[j], gsems[j])

        def g_wait(j):
            pltpu.make_async_copy(p_hbm.at[batch_ds(0)], bufs[j], gsems[j]).wait()

        def s_start(b, j):
            pltpu.async_copy(bufs[j], acc.at[idx.at[b]], ssems[j], add=True)

        def s_wait(j):
            pltpu.make_async_copy(bufs[j], acc.at[idx.at[0]], ssems[j]).wait()

        for j in range(NRING):
            g_start(j, j)
        pltpu.sync_copy(zeros_hbm.at[nds], acc.at[nds])
        plsc.subcore_barrier()

        def body(g, carry):
            base = NRING * g
            for j in range(NRING):
                g_wait(j)
                s_start(base + j, j)

            @pl.when(g < ngrp - 1)
            def _():
                for j in range(NRING):
                    s_wait(j)
                    g_start(base + NRING + j, j)

            return carry

        lax.fori_loop(0, ngrp, body, 0)
        for j in range(NRING):
            s_wait(j)
        plsc.subcore_barrier()
        pltpu.sync_copy(acc.at[nds], out_hbm.at[nds])
        plsc.subcore_barrier()

    @pl.when(c == 0)
    def _():
        do_chunk(p0, o0)
        do_chunk(p1, o1)

    @pl.when(c == 1)
    def _():
        do_chunk(p2, o2)
        do_chunk(p3, o3)


def _sc_scatter(p0, p1, p2, p3, zeros, src3d):
    mesh = plsc.VectorSubcoreMesh(core_axis_name="c", subcore_axis_name="s")
    fn = pl.kernel(
        _sc_body,
        out_type=[jax.ShapeDtypeStruct((NPAD, DA), jnp.float32)] * 4,
        mesh=mesh,
        scratch_types=(
            [pltpu.VMEM((BATCH, DA), jnp.float32)] * NRING
            + [pltpu.VMEM((NBATCH, BATCH), jnp.int32),
               pltpu.VMEM_SHARED((NPAD, DA), jnp.float32)]
            + [pltpu.SemaphoreType.DMA] * (2 * NRING)
        ),
    )
    return fn(p0, p1, p2, p3, zeros, src3d)


def _readout_body(a0u, a0v, a1u, a1v, a2u, a2v, a3u, a3v, wvt,
                  aa, y0, y1, y2):
    aa[...] = a0u[...] + a0v[...]
    y0[...] = jnp.dot(a1u[...] + a1v[...], wvt[...],
                      preferred_element_type=jnp.float32)
    y1[...] = jnp.dot(a2u[...] + a2v[...], wvt[...],
                      preferred_element_type=jnp.float32)
    y2[...] = jnp.dot(a3u[...] + a3v[...], wvt[...],
                      preferred_element_type=jnp.float32)


def _readout(accs_u, accs_v, wvt):
    bn = 632
    node = pl.BlockSpec((bn, DA), lambda i: (i, 0))
    wfull = pl.BlockSpec((DA, DA), lambda i: (0, 0))
    args = []
    for u, v in zip(accs_u, accs_v):
        args += [u, v]
    return pl.pallas_call(
        _readout_body,
        grid=(NPAD // bn,),
        in_specs=[node] * 8 + [wfull],
        out_specs=[node, node, node, node],
        out_shape=[jax.ShapeDtypeStruct((NPAD, DA), jnp.float32)] * 4,
    )(*args, wvt)


def kernel(graph, r_ij, W0, b0, W1, b1, W2, b2, W3, Wv):
    rt = r_ij.T
    rt1 = rt[:, :EPAD_H]
    rt2 = jnp.pad(rt[:, EPAD_H:], ((0, 0), (0, 2 * EPAD_H - NEDGES)))
    src = graph[0]
    src1 = src[:EPAD_H].reshape(16, NBATCH, BATCH)
    src2 = jnp.pad(src[EPAD_H:], (0, 2 * EPAD_H - NEDGES),
                   constant_values=NNODES).reshape(16, NBATCH, BATCH)
    wargs = (W0.T, b0.reshape(1, DA), W1.T, b1.reshape(1, DA),
             W2.T, b2.reshape(1, DA), W3.T)
    zeros = jnp.zeros((NPAD, DA), jnp.float32)
    pu = _edge_mlp(rt1, *wargs)
    au = _sc_scatter(*pu, zeros, src1)
    pv = _edge_mlp(rt2, *wargs)
    av = _sc_scatter(*pv, zeros, src2)
    aa, y0, y1, y2 = _readout(au, av, Wv.T)
    out_v = jnp.stack([y0, y1, y2], axis=-1)[:NNODES]
    return aa[:NNODES], out_v


# speedup vs baseline: 45.0267x; 1.0320x over previous
import jax
import jax.numpy as jnp
from jax import lax
from jax.experimental import pallas as pl
from jax.experimental.pallas import tpu as pltpu
from jax.experimental.pallas import tpu_sc as plsc

R0C = 5.0
NNODES = 10000
NPAD = 10112
NEDGES = 160000
DA = 128

BATCH = 80
NBATCH = 64
EPAD_H = 16 * NBATCH * BATCH
ROWS_PER_TILE = NPAD // 16

BE = 2048


def _leaky(x):
    return jnp.maximum(x, 0.1 * x)


def _edge_body(rt, w0t, b0, w1t, b1, w2t, b2, w3t,
               p0, p1, p2, p3):
    x = rt[0:1, :]
    y = rt[1:2, :]
    z = rt[2:3, :]
    n2 = x * x + y * y + z * z
    xr = jnp.sqrt(n2 + 1e-12) * (1.0 / R0C)
    centers = lax.broadcasted_iota(jnp.int32, (8, 1), 0).astype(jnp.float32) * (1.0 / 7.0)
    d = xr - centers
    enc = jnp.transpose(jnp.exp(-32.0 * d * d), (1, 0))
    h = jnp.dot(enc, w0t[...], preferred_element_type=jnp.float32) + b0[...]
    h = _leaky(jnp.dot(h, w1t[...], preferred_element_type=jnp.float32) + b1[...])
    h = _leaky(jnp.dot(h, w2t[...], preferred_element_type=jnp.float32) + b2[...])
    rad = jnp.dot(h, w3t[...], preferred_element_type=jnp.float32)
    s = 7.0 / R0C
    inv = lax.rsqrt(1.0 + n2 * (s * s))
    srow = jnp.concatenate(
        [x * (s * inv), y * (s * inv), z * (s * inv),
         jnp.zeros((5, x.shape[1]), jnp.float32)], axis=0)
    scol = jnp.transpose(srow, (1, 0))
    p0[...] = rad
    p1[...] = rad * scol[:, 0:1]
    p2[...] = rad * scol[:, 1:2]
    p3[...] = rad * scol[:, 2:3]


def _edge_mlp(rt, w0t, b0, w1t, b1, w2t, b2, w3t):
    grid = EPAD_H // BE
    rspec = pl.BlockSpec((3, BE), lambda i: (0, i))
    full = lambda a: pl.BlockSpec(a.shape, lambda i: (0,) * a.ndim)
    out = pl.BlockSpec((BE, DA), lambda i: (i, 0))
    return pl.pallas_call(
        _edge_body,
        grid=(grid,),
        in_specs=[rspec,
                  full(w0t), full(b0), full(w1t), full(b1),
                  full(w2t), full(b2), full(w3t)],
        out_specs=[out, out, out, out],
        out_shape=[jax.ShapeDtypeStruct((EPAD_H, DA), jnp.float32)] * 4,
    )(rt, w0t, b0, w1t, b1, w2t, b2, w3t)


NRING = 4


def _sc_body(p0, p1, p2, p3, i0, i1, i2, i3, src3d,
             o0, o1, o2, o3, buf0, buf1, buf2, buf3, idx, acc,
             gs0, gs1, gs2, gs3, ss0, ss1, ss2, ss3):
    bufs = (buf0, buf1, buf2, buf3)
    gsems = (gs0, gs1, gs2, gs3)
    ssems = (ss0, ss1, ss2, ss3)
    c = lax.axis_index("c")
    s = lax.axis_index("s")
    pltpu.sync_copy(src3d.at[s], idx)
    nds = pl.ds(s * ROWS_PER_TILE, ROWS_PER_TILE)
    ngrp = NBATCH // NRING

    def do_chunk(p_hbm, init_hbm, out_hbm):
        def batch_ds(b):
            return pl.ds((s * NBATCH + b) * BATCH, BATCH)

        def g_start(b, j):
            pltpu.async_copy(p_hbm.at[batch_ds(b)], bufs[j], gsems[j])

        def g_wait(j):
            pltpu.make_async_copy(p_hbm.at[batch_ds(0)], bufs[j], gsems[j]).wait()

        def s_start(b, j):
            pltpu.async_copy(bufs[j], acc.at[idx.at[b]], ssems[j], add=True)

        def s_wait(j):
            pltpu.make_async_copy(bufs[j], acc.at[idx.at[0]], ssems[j]).wait()

        for j in range(NRING):
            g_start(j, j)
        pltpu.sync_copy(init_hbm.at[nds], acc.at[nds])
        plsc.subcore_barrier()

        def body(g, carry):
            base = NRING * g
            for j in range(NRING):
                g_wait(j)
                s_start(base + j, j)

            @pl.when(g < ngrp - 1)
            def _():
                for j in range(NRING):
                    s_wait(j)
                    g_start(base + NRING + j, j)

            return carry

        lax.fori_loop(0, ngrp, body, 0)
        for j in range(NRING):
            s_wait(j)
        plsc.subcore_barrier()
        pltpu.sync_copy(acc.at[nds], out_hbm.at[nds])
        plsc.subcore_barrier()

    @pl.when(c == 0)
    def _():
        do_chunk(p0, i0, o0)
        do_chunk(p1, i1, o1)

    @pl.when(c == 1)
    def _():
        do_chunk(p2, i2, o2)
        do_chunk(p3, i3, o3)


def _sc_scatter(p0, p1, p2, p3, i0, i1, i2, i3, src3d):
    mesh = plsc.VectorSubcoreMesh(core_axis_name="c", subcore_axis_name="s")
    fn = pl.kernel(
        _sc_body,
        out_type=[jax.ShapeDtypeStruct((NPAD, DA), jnp.float32)] * 4,
        mesh=mesh,
        scratch_types=(
            [pltpu.VMEM((BATCH, DA), jnp.float32)] * NRING
            + [pltpu.VMEM((NBATCH, BATCH), jnp.int32),
               pltpu.VMEM_SHARED((NPAD, DA), jnp.float32)]
            + [pltpu.SemaphoreType.DMA] * (2 * NRING)
        ),
    )
    return fn(p0, p1, p2, p3, i0, i1, i2, i3, src3d)


def _readout_body(a1, a2, a3, wvt, y0, y1, y2):
    y0[...] = jnp.dot(a1[...], wvt[...], preferred_element_type=jnp.float32)
    y1[...] = jnp.dot(a2[...], wvt[...], preferred_element_type=jnp.float32)
    y2[...] = jnp.dot(a3[...], wvt[...], preferred_element_type=jnp.float32)


def _readout(a1, a2, a3, wvt):
    bn = 632
    node = pl.BlockSpec((bn, DA), lambda i: (i, 0))
    wfull = pl.BlockSpec((DA, DA), lambda i: (0, 0))
    return pl.pallas_call(
        _readout_body,
        grid=(NPAD // bn,),
        in_specs=[node, node, node, wfull],
        out_specs=[node, node, node],
        out_shape=[jax.ShapeDtypeStruct((NPAD, DA), jnp.float32)] * 3,
    )(a1, a2, a3, wvt)


def kernel(graph, r_ij, W0, b0, W1, b1, W2, b2, W3, Wv):
    rt = r_ij.T
    rt1 = rt[:, :EPAD_H]
    rt2 = jnp.pad(rt[:, EPAD_H:], ((0, 0), (0, 2 * EPAD_H - NEDGES)))
    src = graph[0]
    src1 = src[:EPAD_H].reshape(16, NBATCH, BATCH)
    src2 = jnp.pad(src[EPAD_H:], (0, 2 * EPAD_H - NEDGES),
                   constant_values=NNODES).reshape(16, NBATCH, BATCH)
    wargs = (W0.T, b0.reshape(1, DA), W1.T, b1.reshape(1, DA),
             W2.T, b2.reshape(1, DA), W3.T)
    zeros = jnp.zeros((NPAD, DA), jnp.float32)
    pu = _edge_mlp(rt1, *wargs)
    au = _sc_scatter(*pu, zeros, zeros, zeros, zeros, src1)
    pv = _edge_mlp(rt2, *wargs)
    a0, a1, a2, a3 = _sc_scatter(*pv, *au, src2)
    y0, y1, y2 = _readout(a1, a2, a3, Wv.T)
    out_v = jnp.stack([y0, y1, y2], axis=-1)[:NNODES]
    return a0[:NNODES], out_v
